# baseline (device time: 39528 ns/iter reference)
import jax
import jax.numpy as jnp
from jax import lax
from jax.experimental import pallas as pl
from jax.experimental.pallas import tpu as pltpu

N_DEV = 8
SQ = 512
D = 1024
DH = 128
HQ_LOC = 8
KV_LOC = 2
SCALE = 0.08838834764831843

_MASKS = (1, 3, 4)
HALF = 256
_PARTS = ((0, (0, 1, 2)), (HALF, (2, 0, 1)))
_RSBUF_OFF = (((0, 128), (128, 64), (192, 64)),
              ((256, 128), (384, 64), (448, 64)))


def kernel(x, Wq, Wo, Wk, Wv):
    my = lax.axis_index("i")
    wk_loc = lax.dynamic_slice(Wk, (0, my * (KV_LOC * DH)), (D, KV_LOC * DH))
    wv_loc = lax.dynamic_slice(Wv, (0, my * (KV_LOC * DH)), (D, KV_LOC * DH))

    def body(x_ref, wq_ref, wo_ref, wk_ref, wv_ref, out_ref,
             p_ref, rsbuf_ref, send_sems, recv_sems):
        my_i = lax.axis_index("i")
        q4 = my_i % 4
        bz = my_i // 4
        by = q4 // 2
        bx = (q4 % 2) ^ by
        bits = (bx, by, bz)
        partners = tuple((my_i ^ m) for m in _MASKS)

        barrier_sem = pltpu.get_barrier_semaphore()
        for nbr in partners:
            pl.semaphore_signal(
                barrier_sem, inc=1,
                device_id=(nbr,), device_id_type=pl.DeviceIdType.MESH,
            )

        xv = x_ref[0, :, :].astype(jnp.bfloat16)
        km = jnp.dot(xv, wk_ref[...].astype(jnp.bfloat16),
                     preferred_element_type=jnp.float32)
        vm = jnp.dot(xv, wv_ref[...].astype(jnp.bfloat16),
                     preferred_element_type=jnp.float32)
        wqb = wq_ref[...].astype(jnp.bfloat16)
        wob = wo_ref[...].astype(jnp.bfloat16)

        def compute_half(base):
            xh = xv[base:base + HALF, :]
            qm = jnp.dot(xh, wqb, preferred_element_type=jnp.float32)
            o_heads = []
            for h in range(HQ_LOC):
                qh = qm[:, h * DH:(h + 1) * DH].astype(jnp.bfloat16)
                kv = h // 4
                kh = km[:, kv * DH:(kv + 1) * DH].astype(jnp.bfloat16)
                vh = vm[:, kv * DH:(kv + 1) * DH].astype(jnp.bfloat16)
                s = jnp.dot(qh, kh.T, preferred_element_type=jnp.float32) * SCALE
                m = jnp.max(s, axis=-1, keepdims=True)
                p = jnp.exp(s - m).astype(jnp.bfloat16)
                l = jnp.sum(p, axis=-1, keepdims=True, dtype=jnp.float32)
                o_heads.append(
                    jnp.dot(p, vh, preferred_element_type=jnp.float32) / l
                )
            o_half = jnp.concatenate(o_heads, axis=1).astype(jnp.bfloat16)
            p_ref[base:base + HALF, :] = jnp.dot(
                o_half, wob, preferred_element_type=jnp.float32
            ).astype(jnp.bfloat16)

        off = [jnp.int32(0), jnp.int32(0)]
        size = [HALF, HALF]
        pending = [None, None]

        def red_start(ip, s):
            base, dims = _PARTS[ip]
            b = bits[dims[s]]
            if s < 2:
                seg = size[ip] // 2
                send_off = base + off[ip] + (1 - b) * seg
            else:
                seg = size[ip]
                send_off = base + off[ip]
            buf_off, _ = _RSBUF_OFF[ip][s]
            rdma = pltpu.make_async_remote_copy(
                src_ref=p_ref.at[pl.ds(send_off, seg), :],
                dst_ref=rsbuf_ref.at[pl.ds(buf_off, seg), :],
                send_sem=send_sems.at[ip * 5 + s],
                recv_sem=recv_sems.at[ip * 5 + s],
                device_id=(partners[dims[s]],),
                device_id_type=pl.DeviceIdType.MESH,
            )
            rdma.start()
            pending[ip] = ("red", rdma, s)

        def ag_start(ip, s):
            base, dims = _PARTS[ip]
            seg = size[ip]
            src = p_ref.at[pl.ds(base + off[ip], seg), :]
            rdma = pltpu.make_async_remote_copy(
                src_ref=src,
                dst_ref=src,
                send_sem=send_sems.at[ip * 5 + s],
                recv_sem=recv_sems.at[ip * 5 + s],
                device_id=(partners[dims[4 - s]],),
                device_id_type=pl.DeviceIdType.MESH,
            )
            rdma.start()
            pending[ip] = ("ag", rdma, s)

        def finish(ip):
            if pending[ip] is None:
                return
            kind, rdma, s = pending[ip]
            pending[ip] = None
            base, dims = _PARTS[ip]
            rdma.wait()
            if kind == "red":
                b = bits[dims[s]]
                buf_off, _ = _RSBUF_OFF[ip][s]
                if s < 2:
                    seg = size[ip] // 2
                    keep = off[ip] + b * seg
                    size[ip] = seg
                    off[ip] = keep
                else:
                    seg = size[ip]
                    keep = off[ip]
                p_ref[pl.ds(base + keep, seg), :] = (
                    p_ref[pl.ds(base + keep, seg), :]
                    + rsbuf_ref[pl.ds(buf_off, seg), :]
                )
            else:
                b = bits[dims[4 - s]]
                size2 = size[ip] * 2
                off[ip] = off[ip] - b * size[ip]
                size[ip] = size2

        compute_half(0)
        pl.semaphore_wait(barrier_sem, 3)
        red_start(0, 0)
        compute_half(HALF)
        red_start(1, 0)

        for step in range(1, 5):
            for ip in range(2):
                finish(ip)
                if step < 3:
                    red_start(ip, step)
                else:
                    ag_start(ip, step)
        for ip in range(2):
            finish(ip)

        out_ref[0, :, :] = p_ref[...]

    out = pl.pallas_call(
        body,
        out_shape=jax.ShapeDtypeStruct((1, SQ, D), jnp.bfloat16),
        in_specs=[pl.BlockSpec(memory_space=pltpu.VMEM)] * 5,
        out_specs=pl.BlockSpec(memory_space=pltpu.VMEM),
        scratch_shapes=[
            pltpu.VMEM((SQ, D), jnp.bfloat16),
            pltpu.VMEM((SQ, D), jnp.bfloat16),
            pltpu.SemaphoreType.DMA((10,)),
            pltpu.SemaphoreType.DMA((10,)),
        ],
        compiler_params=pltpu.CompilerParams(collective_id=0),
    )(x, Wq, Wo, wk_loc, wv_loc)
    return out


# device time: 34431 ns/iter; 1.1480x vs baseline; 1.1480x over previous
import jax
import jax.numpy as jnp
from jax import lax
from jax.experimental import pallas as pl
from jax.experimental.pallas import tpu as pltpu

N_DEV = 8
SQ = 512
D = 1024
DH = 128
HQ_LOC = 8
KV_LOC = 2
SCALE = 0.08838834764831843

_MASKS = (1, 3, 4)
HALF = 256
_PARTS = ((0, (0, 1, 2)), (HALF, (2, 0, 1)))
_RSBUF_OFF = (((0, 128), (128, 64), (192, 64)),
              ((256, 128), (384, 64), (448, 64)))


def kernel(x, Wq, Wo, Wk, Wv):
    my = lax.axis_index("i")
    xb = x.astype(jnp.bfloat16)
    wqb = Wq.astype(jnp.bfloat16)
    wob = Wo.astype(jnp.bfloat16)
    wk_loc = lax.dynamic_slice(
        Wk, (0, my * (KV_LOC * DH)), (D, KV_LOC * DH)).astype(jnp.bfloat16)
    wv_loc = lax.dynamic_slice(
        Wv, (0, my * (KV_LOC * DH)), (D, KV_LOC * DH)).astype(jnp.bfloat16)

    def body(x_ref, wq_ref, wo_ref, wk_ref, wv_ref, out_ref,
             p_ref, rsbuf_ref, send_sems, recv_sems):
        my_i = lax.axis_index("i")
        q4 = my_i % 4
        bz = my_i // 4
        by = q4 // 2
        bx = (q4 % 2) ^ by
        bits = (bx, by, bz)
        partners = tuple((my_i ^ m) for m in _MASKS)

        barrier_sem = pltpu.get_barrier_semaphore()
        for nbr in partners:
            pl.semaphore_signal(
                barrier_sem, inc=1,
                device_id=(nbr,), device_id_type=pl.DeviceIdType.MESH,
            )

        xv = x_ref[0, :, :]
        qm = jnp.dot(xv, wq_ref[...], preferred_element_type=jnp.float32)
        km = jnp.dot(xv, wk_ref[...], preferred_element_type=jnp.float32)
        vm = jnp.dot(xv, wv_ref[...], preferred_element_type=jnp.float32)

        o_heads = []
        for h in range(HQ_LOC):
            qh = qm[:, h * DH:(h + 1) * DH].astype(jnp.bfloat16)
            kv = h // 4
            kh = km[:, kv * DH:(kv + 1) * DH].astype(jnp.bfloat16)
            vh = vm[:, kv * DH:(kv + 1) * DH].astype(jnp.bfloat16)
            s = jnp.dot(qh, kh.T, preferred_element_type=jnp.float32) * SCALE
            m = jnp.max(s, axis=-1, keepdims=True)
            p = jnp.exp(s - m).astype(jnp.bfloat16)
            l = jnp.sum(p, axis=-1, keepdims=True, dtype=jnp.float32)
            o_heads.append(
                jnp.dot(p, vh, preferred_element_type=jnp.float32) / l
            )
        o_loc = jnp.concatenate(o_heads, axis=1).astype(jnp.bfloat16)

        p_ref[...] = jnp.dot(
            o_loc, wo_ref[...], preferred_element_type=jnp.float32,
        ).astype(jnp.bfloat16)

        pl.semaphore_wait(barrier_sem, 3)

        off = [jnp.int32(0), jnp.int32(0)]
        size = [HALF, HALF]
        pending = [None, None]

        def red_start(ip, s):
            base, dims = _PARTS[ip]
            b = bits[dims[s]]
            if s < 2:
                seg = size[ip] // 2
                send_off = base + off[ip] + (1 - b) * seg
            else:
                seg = size[ip]
                send_off = base + off[ip]
            buf_off, _ = _RSBUF_OFF[ip][s]
            rdma = pltpu.make_async_remote_copy(
                src_ref=p_ref.at[pl.ds(send_off, seg), :],
                dst_ref=rsbuf_ref.at[pl.ds(buf_off, seg), :],
                send_sem=send_sems.at[ip * 5 + s],
                recv_sem=recv_sems.at[ip * 5 + s],
                device_id=(partners[dims[s]],),
                device_id_type=pl.DeviceIdType.MESH,
            )
            rdma.start()
            pending[ip] = ("red", rdma, s)

        def ag_start(ip, s):
            base, dims = _PARTS[ip]
            seg = size[ip]
            src = p_ref.at[pl.ds(base + off[ip], seg), :]
            rdma = pltpu.make_async_remote_copy(
                src_ref=src,
                dst_ref=src,
                send_sem=send_sems.at[ip * 5 + s],
                recv_sem=recv_sems.at[ip * 5 + s],
                device_id=(partners[dims[4 - s]],),
                device_id_type=pl.DeviceIdType.MESH,
            )
            rdma.start()
            pending[ip] = ("ag", rdma, s)

        def finish(ip):
            if pending[ip] is None:
                return
            kind, rdma, s = pending[ip]
            pending[ip] = None
            base, dims = _PARTS[ip]
            rdma.wait()
            if kind == "red":
                b = bits[dims[s]]
                buf_off, _ = _RSBUF_OFF[ip][s]
                if s < 2:
                    seg = size[ip] // 2
                    keep = off[ip] + b * seg
                    size[ip] = seg
                    off[ip] = keep
                else:
                    seg = size[ip]
                    keep = off[ip]
                p_ref[pl.ds(base + keep, seg), :] = (
                    p_ref[pl.ds(base + keep, seg), :]
                    + rsbuf_ref[pl.ds(buf_off, seg), :]
                )
            else:
                b = bits[dims[4 - s]]
                size2 = size[ip] * 2
                off[ip] = off[ip] - b * size[ip]
                size[ip] = size2

        for step in range(5):
            for ip in range(2):
                finish(ip)
                if step < 3:
                    red_start(ip, step)
                else:
                    ag_start(ip, step)
        for ip in range(2):
            finish(ip)

        out_ref[0, :, :] = p_ref[...]

    out = pl.pallas_call(
        body,
        out_shape=jax.ShapeDtypeStruct((1, SQ, D), jnp.bfloat16),
        in_specs=[pl.BlockSpec(memory_space=pltpu.VMEM)] * 5,
        out_specs=pl.BlockSpec(memory_space=pltpu.VMEM),
        scratch_shapes=[
            pltpu.VMEM((SQ, D), jnp.bfloat16),
            pltpu.VMEM((SQ, D), jnp.bfloat16),
            pltpu.SemaphoreType.DMA((10,)),
            pltpu.SemaphoreType.DMA((10,)),
        ],
        compiler_params=pltpu.CompilerParams(collective_id=0),
    )(xb, wqb, wob, wk_loc, wv_loc)
    return out


# device time: 31956 ns/iter; 1.2370x vs baseline; 1.0775x over previous
import jax
import jax.numpy as jnp
from jax import lax
from jax.experimental import pallas as pl
from jax.experimental.pallas import tpu as pltpu

N_DEV = 8
SQ = 512
D = 1024
DH = 128
HQ_LOC = 8
KV_LOC = 2
SCALE = 0.08838834764831843

_MASKS = (1, 3, 4)
_PARTS = ((0, 384, (0, 1, 2)), (384, 384, (1, 2, 0)), (768, 256, (2, 0, 1)))
_RSBUF_ROW = ((0, 256), (256, 128), (384, 128))


def kernel(x, Wq, Wo, Wk, Wv):
    my = lax.axis_index("i")
    xb = x.astype(jnp.bfloat16)
    wqb = Wq.astype(jnp.bfloat16)
    wob = Wo.astype(jnp.bfloat16)
    wk_loc = lax.dynamic_slice(
        Wk, (0, my * (KV_LOC * DH)), (D, KV_LOC * DH)).astype(jnp.bfloat16)
    wv_loc = lax.dynamic_slice(
        Wv, (0, my * (KV_LOC * DH)), (D, KV_LOC * DH)).astype(jnp.bfloat16)

    def body(x_ref, wq_ref, wo_ref, wk_ref, wv_ref, out_ref,
             p_ref, rsbuf_ref, send_sems, recv_sems):
        my_i = lax.axis_index("i")
        q4 = my_i % 4
        bz = my_i // 4
        by = q4 // 2
        bx = (q4 % 2) ^ by
        bits = (bx, by, bz)
        partners = tuple((my_i ^ m) for m in _MASKS)

        barrier_sem = pltpu.get_barrier_semaphore()
        for nbr in partners:
            pl.semaphore_signal(
                barrier_sem, inc=1,
                device_id=(nbr,), device_id_type=pl.DeviceIdType.MESH,
            )

        xv = x_ref[0, :, :]
        qm = jnp.dot(xv, wq_ref[...], preferred_element_type=jnp.float32)
        km = jnp.dot(xv, wk_ref[...], preferred_element_type=jnp.float32)
        vm = jnp.dot(xv, wv_ref[...], preferred_element_type=jnp.float32)

        o_heads = []
        for h in range(HQ_LOC):
            qh = qm[:, h * DH:(h + 1) * DH].astype(jnp.bfloat16)
            kv = h // 4
            kh = km[:, kv * DH:(kv + 1) * DH].astype(jnp.bfloat16)
            vh = vm[:, kv * DH:(kv + 1) * DH].astype(jnp.bfloat16)
            s = jnp.dot(qh, kh.T, preferred_element_type=jnp.float32) * SCALE
            m = jnp.max(s, axis=-1, keepdims=True)
            p = jnp.exp(s - m).astype(jnp.bfloat16)
            l = jnp.sum(p, axis=-1, keepdims=True, dtype=jnp.float32)
            o_heads.append(
                jnp.dot(p, vh, preferred_element_type=jnp.float32) / l
            )
        o_loc = jnp.concatenate(o_heads, axis=1).astype(jnp.bfloat16)

        p_ref[...] = jnp.dot(
            o_loc, wo_ref[...], preferred_element_type=jnp.float32,
        ).astype(jnp.bfloat16)

        pl.semaphore_wait(barrier_sem, 3)

        n_parts = len(_PARTS)
        off = [jnp.int32(0) for _ in range(n_parts)]
        size = [SQ] * n_parts
        pending = [None] * n_parts

        def red_start(ip, s):
            col0, ncols, dims = _PARTS[ip]
            b = bits[dims[s]]
            if s < 2:
                seg = size[ip] // 2
                send_off = off[ip] + (1 - b) * seg
            else:
                seg = size[ip]
                send_off = off[ip]
            buf_row, _ = _RSBUF_ROW[s]
            rdma = pltpu.make_async_remote_copy(
                src_ref=p_ref.at[pl.ds(send_off, seg), pl.ds(col0, ncols)],
                dst_ref=rsbuf_ref.at[pl.ds(buf_row, seg), pl.ds(col0, ncols)],
                send_sem=send_sems.at[ip * 5 + s],
                recv_sem=recv_sems.at[ip * 5 + s],
                device_id=(partners[dims[s]],),
                device_id_type=pl.DeviceIdType.MESH,
            )
            rdma.start()
            pending[ip] = ("red", rdma, s)

        def ag_start(ip, s):
            col0, ncols, dims = _PARTS[ip]
            seg = size[ip]
            src = p_ref.at[pl.ds(off[ip], seg), pl.ds(col0, ncols)]
            rdma = pltpu.make_async_remote_copy(
                src_ref=src,
                dst_ref=src,
                send_sem=send_sems.at[ip * 5 + s],
                recv_sem=recv_sems.at[ip * 5 + s],
                device_id=(partners[dims[4 - s]],),
                device_id_type=pl.DeviceIdType.MESH,
            )
            rdma.start()
            pending[ip] = ("ag", rdma, s)

        def finish(ip):
            if pending[ip] is None:
                return
            kind, rdma, s = pending[ip]
            pending[ip] = None
            col0, ncols, dims = _PARTS[ip]
            rdma.wait()
            if kind == "red":
                b = bits[dims[s]]
                buf_row, _ = _RSBUF_ROW[s]
                if s < 2:
                    seg = size[ip] // 2
                    keep = off[ip] + b * seg
                    size[ip] = seg
                    off[ip] = keep
                else:
                    seg = size[ip]
                    keep = off[ip]
                cols = pl.ds(col0, ncols)
                p_ref[pl.ds(keep, seg), cols] = (
                    p_ref[pl.ds(keep, seg), cols]
                    + rsbuf_ref[pl.ds(buf_row, seg), cols]
                )
            else:
                b = bits[dims[4 - s]]
                size2 = size[ip] * 2
                off[ip] = off[ip] - b * size[ip]
                size[ip] = size2

        for step in range(5):
            for ip in range(n_parts):
                finish(ip)
                if step < 3:
                    red_start(ip, step)
                else:
                    ag_start(ip, step)
        for ip in range(n_parts):
            finish(ip)

        out_ref[0, :, :] = p_ref[...]

    out = pl.pallas_call(
        body,
        out_shape=jax.ShapeDtypeStruct((1, SQ, D), jnp.bfloat16),
        in_specs=[pl.BlockSpec(memory_space=pltpu.VMEM)] * 5,
        out_specs=pl.BlockSpec(memory_space=pltpu.VMEM),
        scratch_shapes=[
            pltpu.VMEM((SQ, D), jnp.bfloat16),
            pltpu.VMEM((SQ, D), jnp.bfloat16),
            pltpu.SemaphoreType.DMA((15,)),
            pltpu.SemaphoreType.DMA((15,)),
        ],
        compiler_params=pltpu.CompilerParams(collective_id=0),
    )(xb, wqb, wob, wk_loc, wv_loc)
    return out


# device time: 31797 ns/iter; 1.2431x vs baseline; 1.0050x over previous
import jax
import jax.numpy as jnp
from jax import lax
from jax.experimental import pallas as pl
from jax.experimental.pallas import tpu as pltpu

N_DEV = 8
SQ = 512
D = 1024
DH = 128
HQ_LOC = 8
KV_LOC = 2
SCALE = 0.08838834764831843

_MASKS = (1, 3, 4)
_PARTS = ((0, 384, (0, 1, 2)), (384, 384, (1, 2, 0)), (768, 256, (2, 0, 1)))
_RSBUF_ROW = ((0, 256), (256, 128), (384, 128))


def kernel(x, Wq, Wo, Wk, Wv):
    my = lax.axis_index("i")
    xb = x.astype(jnp.bfloat16)
    wqb = Wq.astype(jnp.bfloat16)
    wob = Wo.astype(jnp.bfloat16)
    wk_loc = lax.dynamic_slice(
        Wk, (0, my * (KV_LOC * DH)), (D, KV_LOC * DH)).astype(jnp.bfloat16)
    wv_loc = lax.dynamic_slice(
        Wv, (0, my * (KV_LOC * DH)), (D, KV_LOC * DH)).astype(jnp.bfloat16)

    def body(x_ref, wq_ref, wo_ref, wk_ref, wv_ref, out_ref,
             p_ref, rsbuf_ref, send_sems, recv_sems):
        my_i = lax.axis_index("i")
        q4 = my_i % 4
        bz = my_i // 4
        by = q4 // 2
        bx = (q4 % 2) ^ by
        bits = (bx, by, bz)
        partners = tuple((my_i ^ m) for m in _MASKS)

        barrier_sem = pltpu.get_barrier_semaphore()
        for nbr in partners:
            pl.semaphore_signal(
                barrier_sem, inc=1,
                device_id=(nbr,), device_id_type=pl.DeviceIdType.MESH,
            )

        xv = x_ref[0, :, :]
        qm = jnp.dot(xv, wq_ref[...], preferred_element_type=jnp.float32)
        km = jnp.dot(xv, wk_ref[...], preferred_element_type=jnp.float32)
        vm = jnp.dot(xv, wv_ref[...], preferred_element_type=jnp.float32)

        o_heads = []
        for h in range(HQ_LOC):
            qh = qm[:, h * DH:(h + 1) * DH].astype(jnp.bfloat16)
            kv = h // 4
            kh = km[:, kv * DH:(kv + 1) * DH].astype(jnp.bfloat16)
            vh = vm[:, kv * DH:(kv + 1) * DH].astype(jnp.bfloat16)
            s = jnp.dot(qh, kh.T, preferred_element_type=jnp.float32) * SCALE
            m = jnp.max(s, axis=-1, keepdims=True)
            p = jnp.exp(s - m).astype(jnp.bfloat16)
            l = jnp.sum(p, axis=-1, keepdims=True, dtype=jnp.float32)
            o_heads.append(
                jnp.dot(p, vh, preferred_element_type=jnp.float32) / l
            )
        o_loc = jnp.concatenate(o_heads, axis=1).astype(jnp.bfloat16)

        n_parts = len(_PARTS)
        off = [jnp.int32(0) for _ in range(n_parts)]
        size = [SQ] * n_parts
        pending = [None] * n_parts

        def red_start(ip, s):
            col0, ncols, dims = _PARTS[ip]
            b = bits[dims[s]]
            if s < 2:
                seg = size[ip] // 2
                send_off = off[ip] + (1 - b) * seg
            else:
                seg = size[ip]
                send_off = off[ip]
            buf_row, _ = _RSBUF_ROW[s]
            rdma = pltpu.make_async_remote_copy(
                src_ref=p_ref.at[pl.ds(send_off, seg), pl.ds(col0, ncols)],
                dst_ref=rsbuf_ref.at[pl.ds(buf_row, seg), pl.ds(col0, ncols)],
                send_sem=send_sems.at[ip * 5 + s],
                recv_sem=recv_sems.at[ip * 5 + s],
                device_id=(partners[dims[s]],),
                device_id_type=pl.DeviceIdType.MESH,
            )
            rdma.start()
            pending[ip] = ("red", rdma, s)

        def ag_start(ip, s):
            col0, ncols, dims = _PARTS[ip]
            seg = size[ip]
            src = p_ref.at[pl.ds(off[ip], seg), pl.ds(col0, ncols)]
            rdma = pltpu.make_async_remote_copy(
                src_ref=src,
                dst_ref=src,
                send_sem=send_sems.at[ip * 5 + s],
                recv_sem=recv_sems.at[ip * 5 + s],
                device_id=(partners[dims[4 - s]],),
                device_id_type=pl.DeviceIdType.MESH,
            )
            rdma.start()
            pending[ip] = ("ag", rdma, s)

        def finish(ip):
            if pending[ip] is None:
                return
            kind, rdma, s = pending[ip]
            pending[ip] = None
            col0, ncols, dims = _PARTS[ip]
            rdma.wait()
            if kind == "red":
                b = bits[dims[s]]
                buf_row, _ = _RSBUF_ROW[s]
                if s < 2:
                    seg = size[ip] // 2
                    keep = off[ip] + b * seg
                    size[ip] = seg
                    off[ip] = keep
                else:
                    seg = size[ip]
                    keep = off[ip]
                cols = pl.ds(col0, ncols)
                p_ref[pl.ds(keep, seg), cols] = (
                    p_ref[pl.ds(keep, seg), cols]
                    + rsbuf_ref[pl.ds(buf_row, seg), cols]
                )
            else:
                b = bits[dims[4 - s]]
                size2 = size[ip] * 2
                off[ip] = off[ip] - b * size[ip]
                size[ip] = size2

        for ip in range(n_parts):
            col0, ncols, _ = _PARTS[ip]
            p_ref[:, col0:col0 + ncols] = jnp.dot(
                o_loc, wo_ref[:, col0:col0 + ncols],
                preferred_element_type=jnp.float32,
            ).astype(jnp.bfloat16)
            if ip == 0:
                pl.semaphore_wait(barrier_sem, 3)
            red_start(ip, 0)

        for step in range(1, 5):
            for ip in range(n_parts):
                finish(ip)
                if step < 3:
                    red_start(ip, step)
                else:
                    ag_start(ip, step)
        for ip in range(n_parts):
            finish(ip)

        out_ref[0, :, :] = p_ref[...]

    out = pl.pallas_call(
        body,
        out_shape=jax.ShapeDtypeStruct((1, SQ, D), jnp.bfloat16),
        in_specs=[pl.BlockSpec(memory_space=pltpu.VMEM)] * 5,
        out_specs=pl.BlockSpec(memory_space=pltpu.VMEM),
        scratch_shapes=[
            pltpu.VMEM((SQ, D), jnp.bfloat16),
            pltpu.VMEM((SQ, D), jnp.bfloat16),
            pltpu.SemaphoreType.DMA((15,)),
            pltpu.SemaphoreType.DMA((15,)),
        ],
        compiler_params=pltpu.CompilerParams(collective_id=0),
    )(xb, wqb, wob, wk_loc, wv_loc)
    return out


# device time: 31458 ns/iter; 1.2565x vs baseline; 1.0108x over previous
import jax
import jax.numpy as jnp
from jax import lax
from jax.experimental import pallas as pl
from jax.experimental.pallas import tpu as pltpu

N_DEV = 8
SQ = 512
D = 1024
DH = 128
HQ_LOC = 8
KV_LOC = 2
SCALE = 0.08838834764831843

_MASKS = (1, 3, 4)
_PARTS = ((0, 384, (0, 1, 2)), (384, 384, (1, 2, 0)), (768, 256, (2, 0, 1)))
_RSBUF_ROW = ((0, 256), (256, 128), (384, 128))


def kernel(x, Wq, Wo, Wk, Wv):
    my = lax.axis_index("i")
    xb = x.astype(jnp.bfloat16)
    wqb = Wq.astype(jnp.bfloat16)
    wob = Wo.astype(jnp.bfloat16)
    wk_loc = lax.dynamic_slice(
        Wk, (0, my * (KV_LOC * DH)), (D, KV_LOC * DH)).astype(jnp.bfloat16)
    wv_loc = lax.dynamic_slice(
        Wv, (0, my * (KV_LOC * DH)), (D, KV_LOC * DH)).astype(jnp.bfloat16)

    def body(x_ref, wq_ref, wo_ref, wk_ref, wv_ref, out_ref,
             rsbuf_ref, send_sems, recv_sems):
        p_ref = out_ref.at[0]
        my_i = lax.axis_index("i")
        q4 = my_i % 4
        bz = my_i // 4
        by = q4 // 2
        bx = (q4 % 2) ^ by
        bits = (bx, by, bz)
        partners = tuple((my_i ^ m) for m in _MASKS)

        barrier_sem = pltpu.get_barrier_semaphore()
        for nbr in partners:
            pl.semaphore_signal(
                barrier_sem, inc=1,
                device_id=(nbr,), device_id_type=pl.DeviceIdType.MESH,
            )

        xv = x_ref[0, :, :]
        qm = jnp.dot(xv, wq_ref[...], preferred_element_type=jnp.float32)
        km = jnp.dot(xv, wk_ref[...], preferred_element_type=jnp.float32)
        vm = jnp.dot(xv, wv_ref[...], preferred_element_type=jnp.float32)

        o_heads = []
        for h in range(HQ_LOC):
            qh = qm[:, h * DH:(h + 1) * DH].astype(jnp.bfloat16)
            kv = h // 4
            kh = km[:, kv * DH:(kv + 1) * DH].astype(jnp.bfloat16)
            vh = vm[:, kv * DH:(kv + 1) * DH].astype(jnp.bfloat16)
            s = jnp.dot(qh, kh.T, preferred_element_type=jnp.float32) * SCALE
            p = jnp.exp(s).astype(jnp.bfloat16)
            l = jnp.sum(p, axis=-1, keepdims=True, dtype=jnp.float32)
            o_heads.append(
                jnp.dot(p, vh, preferred_element_type=jnp.float32) / l
            )
        o_loc = jnp.concatenate(o_heads, axis=1).astype(jnp.bfloat16)

        n_parts = len(_PARTS)
        off = [jnp.int32(0) for _ in range(n_parts)]
        size = [SQ] * n_parts
        pending = [None] * n_parts

        def red_start(ip, s):
            col0, ncols, dims = _PARTS[ip]
            b = bits[dims[s]]
            if s < 2:
                seg = size[ip] // 2
                send_off = off[ip] + (1 - b) * seg
            else:
                seg = size[ip]
                send_off = off[ip]
            buf_row, _ = _RSBUF_ROW[s]
            rdma = pltpu.make_async_remote_copy(
                src_ref=p_ref.at[pl.ds(send_off, seg), pl.ds(col0, ncols)],
                dst_ref=rsbuf_ref.at[pl.ds(buf_row, seg), pl.ds(col0, ncols)],
                send_sem=send_sems.at[ip * 5 + s],
                recv_sem=recv_sems.at[ip * 5 + s],
                device_id=(partners[dims[s]],),
                device_id_type=pl.DeviceIdType.MESH,
            )
            rdma.start()
            pending[ip] = ("red", rdma, s)

        def ag_start(ip, s):
            col0, ncols, dims = _PARTS[ip]
            seg = size[ip]
            src = p_ref.at[pl.ds(off[ip], seg), pl.ds(col0, ncols)]
            rdma = pltpu.make_async_remote_copy(
                src_ref=src,
                dst_ref=src,
                send_sem=send_sems.at[ip * 5 + s],
                recv_sem=recv_sems.at[ip * 5 + s],
                device_id=(partners[dims[4 - s]],),
                device_id_type=pl.DeviceIdType.MESH,
            )
            rdma.start()
            pending[ip] = ("ag", rdma, s)

        def finish(ip):
            if pending[ip] is None:
                return
            kind, rdma, s = pending[ip]
            pending[ip] = None
            col0, ncols, dims = _PARTS[ip]
            rdma.wait()
            if kind == "red":
                b = bits[dims[s]]
                buf_row, _ = _RSBUF_ROW[s]
                if s < 2:
                    seg = size[ip] // 2
                    keep = off[ip] + b * seg
                    size[ip] = seg
                    off[ip] = keep
                else:
                    seg = size[ip]
                    keep = off[ip]
                cols = pl.ds(col0, ncols)
                p_ref[pl.ds(keep, seg), cols] = (
                    p_ref[pl.ds(keep, seg), cols]
                    + rsbuf_ref[pl.ds(buf_row, seg), cols]
                )
            else:
                b = bits[dims[4 - s]]
                size2 = size[ip] * 2
                off[ip] = off[ip] - b * size[ip]
                size[ip] = size2

        for ip in range(n_parts):
            col0, ncols, _ = _PARTS[ip]
            p_ref[:, col0:col0 + ncols] = jnp.dot(
                o_loc, wo_ref[:, col0:col0 + ncols],
                preferred_element_type=jnp.float32,
            ).astype(jnp.bfloat16)
            if ip == 0:
                pl.semaphore_wait(barrier_sem, 3)
            red_start(ip, 0)

        for step in range(1, 5):
            for ip in range(n_parts):
                finish(ip)
                if step < 3:
                    red_start(ip, step)
                else:
                    ag_start(ip, step)
        for ip in range(n_parts):
            finish(ip)

    out = pl.pallas_call(
        body,
        out_shape=jax.ShapeDtypeStruct((1, SQ, D), jnp.bfloat16),
        in_specs=[pl.BlockSpec(memory_space=pltpu.VMEM)] * 5,
        out_specs=pl.BlockSpec(memory_space=pltpu.VMEM),
        scratch_shapes=[
            pltpu.VMEM((SQ, D), jnp.bfloat16),
            pltpu.SemaphoreType.DMA((15,)),
            pltpu.SemaphoreType.DMA((15,)),
        ],
        compiler_params=pltpu.CompilerParams(collective_id=0),
    )(xb, wqb, wob, wk_loc, wv_loc)
    return out


# device time: 31056 ns/iter; 1.2728x vs baseline; 1.0129x over previous
import jax
import jax.numpy as jnp
from jax import lax
from jax.experimental import pallas as pl
from jax.experimental.pallas import tpu as pltpu

N_DEV = 8
SQ = 512
D = 1024
DH = 128
HQ_LOC = 8
KV_LOC = 2
SCALE = 0.08838834764831843

_MASKS = (1, 3, 4)
_PARTS = ((0, 384, (0, 1, 2)), (384, 384, (1, 2, 0)), (768, 256, (2, 0, 1)))
_RSBUF_ROW = ((0, 256), (256, 256), (512, 256))


def kernel(x, Wq, Wo, Wk, Wv):
    my = lax.axis_index("i")
    xb = x.astype(jnp.bfloat16)
    wqb = Wq.astype(jnp.bfloat16)
    wob = Wo.astype(jnp.bfloat16)
    wk_loc = lax.dynamic_slice(
        Wk, (0, my * (KV_LOC * DH)), (D, KV_LOC * DH)).astype(jnp.bfloat16)
    wv_loc = lax.dynamic_slice(
        Wv, (0, my * (KV_LOC * DH)), (D, KV_LOC * DH)).astype(jnp.bfloat16)

    def body(x_ref, wq_ref, wo_ref, wk_ref, wv_ref, out_ref,
             rsbuf_ref, send_sems, recv_sems):
        p_ref = out_ref.at[0]
        my_i = lax.axis_index("i")
        q4 = my_i % 4
        bz = my_i // 4
        by = q4 // 2
        bx = (q4 % 2) ^ by
        bits = (bx, by, bz)
        partners = tuple((my_i ^ m) for m in _MASKS)

        barrier_sem = pltpu.get_barrier_semaphore()
        for nbr in partners:
            pl.semaphore_signal(
                barrier_sem, inc=1,
                device_id=(nbr,), device_id_type=pl.DeviceIdType.MESH,
            )

        xv = x_ref[0, :, :]
        qm = jnp.dot(xv, wq_ref[...], preferred_element_type=jnp.float32)
        km = jnp.dot(xv, wk_ref[...], preferred_element_type=jnp.float32)
        vm = jnp.dot(xv, wv_ref[...], preferred_element_type=jnp.float32)

        o_heads = []
        for h in range(HQ_LOC):
            qh = qm[:, h * DH:(h + 1) * DH].astype(jnp.bfloat16)
            kv = h // 4
            kh = km[:, kv * DH:(kv + 1) * DH].astype(jnp.bfloat16)
            vh = vm[:, kv * DH:(kv + 1) * DH].astype(jnp.bfloat16)
            s = lax.dot_general(
                qh, kh, (((1,), (1,)), ((), ())),
                preferred_element_type=jnp.float32) * SCALE
            p = jnp.exp(s).astype(jnp.bfloat16)
            l = jnp.sum(p, axis=-1, keepdims=True, dtype=jnp.float32)
            o_heads.append(
                jnp.dot(p, vh, preferred_element_type=jnp.float32) / l
            )
        o_loc = jnp.concatenate(o_heads, axis=1).astype(jnp.bfloat16)

        n_parts = len(_PARTS)
        off = [jnp.int32(0) for _ in range(n_parts)]
        size = [SQ] * n_parts
        pending = [None] * n_parts

        def red_start(ip, s):
            col0, ncols, dims = _PARTS[ip]
            b = bits[dims[s]]
            if s < 1:
                seg = size[ip] // 2
                send_off = off[ip] + (1 - b) * seg
            else:
                seg = size[ip]
                send_off = off[ip]
            buf_row, _ = _RSBUF_ROW[s]
            rdma = pltpu.make_async_remote_copy(
                src_ref=p_ref.at[pl.ds(send_off, seg), pl.ds(col0, ncols)],
                dst_ref=rsbuf_ref.at[pl.ds(buf_row, seg), pl.ds(col0, ncols)],
                send_sem=send_sems.at[ip * 5 + s],
                recv_sem=recv_sems.at[ip * 5 + s],
                device_id=(partners[dims[s]],),
                device_id_type=pl.DeviceIdType.MESH,
            )
            rdma.start()
            pending[ip] = ("red", rdma, s)

        def ag_start(ip, s):
            col0, ncols, dims = _PARTS[ip]
            seg = size[ip]
            src = p_ref.at[pl.ds(off[ip], seg), pl.ds(col0, ncols)]
            rdma = pltpu.make_async_remote_copy(
                src_ref=src,
                dst_ref=src,
                send_sem=send_sems.at[ip * 5 + s],
                recv_sem=recv_sems.at[ip * 5 + s],
                device_id=(partners[dims[0]],),
                device_id_type=pl.DeviceIdType.MESH,
            )
            rdma.start()
            pending[ip] = ("ag", rdma, s)

        def finish(ip):
            if pending[ip] is None:
                return
            kind, rdma, s = pending[ip]
            pending[ip] = None
            col0, ncols, dims = _PARTS[ip]
            rdma.wait()
            if kind == "red":
                b = bits[dims[s]]
                buf_row, _ = _RSBUF_ROW[s]
                if s < 1:
                    seg = size[ip] // 2
                    keep = off[ip] + b * seg
                    size[ip] = seg
                    off[ip] = keep
                else:
                    seg = size[ip]
                    keep = off[ip]
                cols = pl.ds(col0, ncols)
                p_ref[pl.ds(keep, seg), cols] = (
                    p_ref[pl.ds(keep, seg), cols]
                    + rsbuf_ref[pl.ds(buf_row, seg), cols]
                )
            else:
                b = bits[dims[0]]
                size2 = size[ip] * 2
                off[ip] = off[ip] - b * size[ip]
                size[ip] = size2

        for ip in range(n_parts):
            col0, ncols, _ = _PARTS[ip]
            p_ref[:, col0:col0 + ncols] = jnp.dot(
                o_loc, wo_ref[:, col0:col0 + ncols],
                preferred_element_type=jnp.float32,
            ).astype(jnp.bfloat16)
            if ip == 0:
                pl.semaphore_wait(barrier_sem, 3)
            red_start(ip, 0)

        for step in range(1, 4):
            for ip in range(n_parts):
                finish(ip)
                if step < 3:
                    red_start(ip, step)
                else:
                    ag_start(ip, step)
        for ip in range(n_parts):
            finish(ip)

    out = pl.pallas_call(
        body,
        out_shape=jax.ShapeDtypeStruct((1, SQ, D), jnp.bfloat16),
        in_specs=[pl.BlockSpec(memory_space=pltpu.VMEM)] * 5,
        out_specs=pl.BlockSpec(memory_space=pltpu.VMEM),
        scratch_shapes=[
            pltpu.VMEM((768, D), jnp.bfloat16),
            pltpu.SemaphoreType.DMA((15,)),
            pltpu.SemaphoreType.DMA((15,)),
        ],
        compiler_params=pltpu.CompilerParams(collective_id=0),
    )(xb, wqb, wob, wk_loc, wv_loc)
    return out
